# baseline (speedup 1.0000x reference)
import functools

import jax
import jax.numpy as jnp
from jax.experimental import pallas as pl
from jax.experimental.pallas import tpu as pltpu


def _gru2_fc_kernel(x_hbm,
                    wih0_ref, whh0_ref, bih0_ref, bhh0_ref,
                    wih1_ref, whh1_ref, bih1_ref, bhh1_ref,
                    fcw_ref, fcb_ref,
                    out_ref,
                    x_buf, wih0t, whh0t, wih1t, whh1t, in_sem,
                    *, T, B, H, D):
    f32 = jnp.float32
    bf16 = jnp.bfloat16
    half = bf16(0.5)

    def dma(t):
        return pltpu.make_async_copy(x_hbm.at[:, pl.ds(t * D, D)],
                                     x_buf.at[t], in_sem.at[t])

    for t in range(T):
        dma(t).start()

    col = jax.lax.broadcasted_iota(jnp.int32, (1, 3 * H), 1)
    scale = jnp.where(col < 2 * H, f32(0.5), f32(1.0))
    wih0t[...] = (wih0_ref[...].T * scale).astype(bf16)
    whh0t[...] = (whh0_ref[...].T * f32(0.5)).astype(bf16)
    wih1t[...] = (wih1_ref[...].T * scale).astype(bf16)
    whh1t[...] = (whh1_ref[...].T * f32(0.5)).astype(bf16)

    rz0 = bih0_ref[:, :2 * H] + bhh0_ref[:, :2 * H]
    bx0 = (scale * jnp.concatenate([rz0, bih0_ref[:, 2 * H:]],
                                   axis=1)).astype(bf16)
    bhn0 = (f32(0.5) * bhh0_ref[:, 2 * H:]).astype(bf16)
    rz1 = bih1_ref[:, :2 * H] + bhh1_ref[:, :2 * H]
    bx1 = (scale * jnp.concatenate([rz1, bih1_ref[:, 2 * H:]],
                                   axis=1)).astype(bf16)
    bhn1 = (f32(0.5) * bhh1_ref[:, 2 * H:]).astype(bf16)

    def gru_step(g, h, whht, bhn):
        gh = jnp.dot(h, whht[...], preferred_element_type=f32).astype(bf16)
        tr = jnp.tanh(g[:, 0 * H:1 * H] + gh[:, 0 * H:1 * H])
        tz = jnp.tanh(g[:, 1 * H:2 * H] + gh[:, 1 * H:2 * H])
        ghn = gh[:, 2 * H:3 * H] + bhn
        n = jnp.tanh(g[:, 2 * H:3 * H] + ghn + tr * ghn)
        return half * ((h + n) + tz * (h - n))

    h0 = jnp.zeros((B, H), bf16)
    h1 = jnp.zeros((B, H), bf16)
    g1 = None
    for t in range(T):
        dma(t).wait()
        xt = x_buf[t].astype(bf16)
        g0 = (jnp.dot(xt, wih0t[...], preferred_element_type=f32)
              .astype(bf16) + bx0)
        h0 = gru_step(g0, h0, whh0t, bhn0)
        if t >= 1:
            h1 = gru_step(g1, h1, whh1t, bhn1)
        g1 = (jnp.dot(h0, wih1t[...], preferred_element_type=f32)
              .astype(bf16) + bx1)
    h1 = gru_step(g1, h1, whh1t, bhn1)

    out_ref[...] = (jax.lax.dot_general(
        h1, fcw_ref[...].astype(bf16), (((1,), (1,)), ((), ())),
        preferred_element_type=f32) + fcb_ref[...]).astype(out_ref.dtype)


def kernel(w_ih_0, w_hh_0, b_ih_0, b_hh_0,
           w_ih_1, w_hh_1, b_ih_1, b_hh_1,
           fc_w, fc_b, x):
    B, T, D = x.shape
    H = w_hh_0.shape[1]
    C = fc_w.shape[0]
    bf16 = jnp.bfloat16

    operands = [x.reshape(B, T * D),
                w_ih_0, w_hh_0, b_ih_0.reshape(1, 3 * H),
                b_hh_0.reshape(1, 3 * H),
                w_ih_1, w_hh_1, b_ih_1.reshape(1, 3 * H),
                b_hh_1.reshape(1, 3 * H),
                fc_w, fc_b.reshape(1, C)]
    in_specs = [pl.BlockSpec(memory_space=pl.ANY)]
    in_specs += [pl.BlockSpec(a.shape, lambda i, nd=a.ndim: (0,) * nd)
                 for a in operands[1:]]

    out = pl.pallas_call(
        functools.partial(_gru2_fc_kernel, T=T, B=B, H=H, D=D),
        out_shape=jax.ShapeDtypeStruct((B, C), jnp.float32),
        grid=(1,),
        in_specs=in_specs,
        out_specs=pl.BlockSpec((B, C), lambda i: (0, 0)),
        scratch_shapes=[
            pltpu.VMEM((T, B, D), jnp.float32),
            pltpu.VMEM((D, 3 * H), bf16),
            pltpu.VMEM((H, 3 * H), bf16),
            pltpu.VMEM((H, 3 * H), bf16),
            pltpu.VMEM((H, 3 * H), bf16),
            pltpu.SemaphoreType.DMA((T,)),
        ],
        compiler_params=pltpu.CompilerParams(
            dimension_semantics=("arbitrary",)),
    )(*operands)
    return out


# speedup vs baseline: 1.7072x; 1.7072x over previous
import functools

import jax
import jax.numpy as jnp
from jax.experimental import pallas as pl
from jax.experimental.pallas import tpu as pltpu


def _gru2_fc_kernel(x_hbm,
                    wih0_ref, whh0_ref, bih0_ref, bhh0_ref,
                    wih1_ref, whh1_ref, bih1_ref, bhh1_ref,
                    fcw_ref, fcb_ref,
                    out_ref,
                    x_buf, wih0t, whh0t, wih1t, whh1t, in_sem,
                    *, T, B, H, D):
    f32 = jnp.float32
    bf16 = jnp.bfloat16
    half = bf16(0.5)

    def dma(t):
        return pltpu.make_async_copy(x_hbm.at[:, t, :], x_buf.at[t],
                                     in_sem.at[t])

    for t in range(T):
        dma(t).start()

    col = jax.lax.broadcasted_iota(jnp.int32, (1, 3 * H), 1)
    scale = jnp.where(col < 2 * H, f32(0.5), f32(1.0))
    wih0t[...] = (wih0_ref[...].T * scale).astype(bf16)
    whh0t[...] = (whh0_ref[...].T * f32(0.5)).astype(bf16)
    wih1t[...] = (wih1_ref[...].T * scale).astype(bf16)
    whh1t[...] = (whh1_ref[...].T * f32(0.5)).astype(bf16)

    bih0 = bih0_ref[...].reshape(1, 3 * H)
    bhh0 = bhh0_ref[...].reshape(1, 3 * H)
    bih1 = bih1_ref[...].reshape(1, 3 * H)
    bhh1 = bhh1_ref[...].reshape(1, 3 * H)
    rz0 = bih0[:, :2 * H] + bhh0[:, :2 * H]
    bx0 = (scale * jnp.concatenate([rz0, bih0[:, 2 * H:]], axis=1)
           ).astype(bf16)
    bhn0 = (f32(0.5) * bhh0[:, 2 * H:]).astype(bf16)
    rz1 = bih1[:, :2 * H] + bhh1[:, :2 * H]
    bx1 = (scale * jnp.concatenate([rz1, bih1[:, 2 * H:]], axis=1)
           ).astype(bf16)
    bhn1 = (f32(0.5) * bhh1[:, 2 * H:]).astype(bf16)

    def gru_step(g, h, whht, bhn):
        gh = jnp.dot(h, whht[...], preferred_element_type=f32).astype(bf16)
        tr = jnp.tanh(g[:, 0 * H:1 * H] + gh[:, 0 * H:1 * H])
        tz = jnp.tanh(g[:, 1 * H:2 * H] + gh[:, 1 * H:2 * H])
        ghn = gh[:, 2 * H:3 * H] + bhn
        n = jnp.tanh(g[:, 2 * H:3 * H] + ghn + tr * ghn)
        return half * ((h + n) + tz * (h - n))

    h0 = jnp.zeros((B, H), bf16)
    h1 = jnp.zeros((B, H), bf16)
    g1 = None
    for t in range(T):
        dma(t).wait()
        xt = x_buf[t].astype(bf16)
        g0 = (jnp.dot(xt, wih0t[...], preferred_element_type=f32)
              .astype(bf16) + bx0)
        h0 = gru_step(g0, h0, whh0t, bhn0)
        if t >= 1:
            h1 = gru_step(g1, h1, whh1t, bhn1)
        g1 = (jnp.dot(h0, wih1t[...], preferred_element_type=f32)
              .astype(bf16) + bx1)
    h1 = gru_step(g1, h1, whh1t, bhn1)

    out_ref[...] = (jax.lax.dot_general(
        h1, fcw_ref[...].astype(bf16), (((1,), (1,)), ((), ())),
        preferred_element_type=f32)
        + fcb_ref[...].reshape(1, fcb_ref.shape[0])).astype(out_ref.dtype)


def kernel(w_ih_0, w_hh_0, b_ih_0, b_hh_0,
           w_ih_1, w_hh_1, b_ih_1, b_hh_1,
           fc_w, fc_b, x):
    B, T, D = x.shape
    H = w_hh_0.shape[1]
    bf16 = jnp.bfloat16
    C = fc_w.shape[0]

    operands = [x, w_ih_0, w_hh_0, b_ih_0, b_hh_0,
                w_ih_1, w_hh_1, b_ih_1, b_hh_1, fc_w, fc_b]
    in_specs = [pl.BlockSpec(memory_space=pl.ANY)]
    in_specs += [pl.BlockSpec(a.shape, lambda i, nd=a.ndim: (0,) * nd)
                 for a in operands[1:]]

    out = pl.pallas_call(
        functools.partial(_gru2_fc_kernel, T=T, B=B, H=H, D=D),
        out_shape=jax.ShapeDtypeStruct((B, C), jnp.float32),
        grid=(1,),
        in_specs=in_specs,
        out_specs=pl.BlockSpec((B, C), lambda i: (0, 0)),
        scratch_shapes=[
            pltpu.VMEM((T, B, D), jnp.float32),
            pltpu.VMEM((D, 3 * H), bf16),
            pltpu.VMEM((H, 3 * H), bf16),
            pltpu.VMEM((H, 3 * H), bf16),
            pltpu.VMEM((H, 3 * H), bf16),
            pltpu.SemaphoreType.DMA((T,)),
        ],
        compiler_params=pltpu.CompilerParams(
            dimension_semantics=("arbitrary",)),
    )(*operands)
    return out
